# baseline (speedup 1.0000x reference)
import functools

import jax
import jax.numpy as jnp
from jax import lax
from jax.experimental import pallas as pl
from jax.experimental.pallas import tpu as pltpu
from jax.experimental.pallas import tpu_sc as plsc

H = 512
W = 512
C = 3
N = 2097152

LANES = 16
NC = 2
NS = 16
NW = NC * NS

CPT = N // NW
CHUNK = 2048
NCHUNK = CPT // CHUNK
GSEG = 128
NSEGE = C * CHUNK // GSEG
NGRP = CHUNK // LANES


def _lane_gather(v, idx):
    dnums = lax.GatherDimensionNumbers(
        offset_dims=(), collapsed_slice_dims=(0,), start_index_map=(0,)
    )
    return lax.gather(
        v, idx[:, None], dnums, (1,),
        mode=lax.GatherScatterMode.PROMISE_IN_BOUNDS,
    )


def _sc_lookup(coords_flat, image_flat):
    mesh = plsc.VectorSubcoreMesh(core_axis_name="c", subcore_axis_name="s")

    @functools.partial(
        pl.kernel,
        mesh=mesh,
        out_type=jax.ShapeDtypeStruct((N * C,), jnp.float32),
        compiler_params=pltpu.CompilerParams(use_tc_tiling_on_sc=False),
        scratch_types=[
            pltpu.VMEM((2 * CHUNK,), jnp.float32),
            pltpu.VMEM((C * CHUNK,), jnp.int32),
            pltpu.VMEM((C * CHUNK,), jnp.float32),
            pltpu.SemaphoreType.DMA,
        ],
    )
    def k(coords_hbm, img_hbm, out_hbm, cbuf, ebuf, obuf, sem):
        wid = lax.axis_index("s") * NC + lax.axis_index("c")
        base = wid * CPT

        lanesv = lax.iota(jnp.int32, LANES)
        mult = jnp.where((lanesv & 1) == 0, W, 1)
        swap = lanesv ^ 1
        epat = (lanesv * 2) & (LANES - 1)
        lowhalf = lanesv < 8

        cpat = []
        rpat = []
        for t in range(C):
            e = lanesv + (LANES * t)
            q = (e * 21846) >> 16
            cpat.append(q)
            rpat.append(e - C * q)

        def chunk_body(g, carry):
            cstart = base + g * CHUNK
            pltpu.sync_copy(
                coords_hbm.at[pl.ds(2 * cstart, 2 * CHUNK)], cbuf
            )

            def grp_body(i, carry):
                for kk in range(4):
                    gi = 4 * i + kk
                    o = 2 * LANES * gi
                    a = cbuf[pl.ds(o, LANES)].astype(jnp.int32)
                    b = cbuf[pl.ds(o + LANES, LANES)].astype(jnp.int32)
                    a = jnp.minimum(jnp.maximum(a, 0), W - 1) * mult
                    b = jnp.minimum(jnp.maximum(b, 0), W - 1) * mult
                    sa = a + _lane_gather(a, swap)
                    sb = b + _lane_gather(b, swap)
                    pix = jnp.where(
                        lowhalf,
                        _lane_gather(sa, epat),
                        _lane_gather(sb, epat),
                    )
                    for t in range(C):
                        ev = C * _lane_gather(pix, cpat[t]) + rpat[t]
                        ebuf[pl.ds(C * LANES * gi + LANES * t, LANES)] = ev
                return carry

            lax.fori_loop(0, NGRP // 4, grp_body, 0)

            copies = [
                pltpu.async_copy(
                    img_hbm.at[ebuf.at[pl.ds(GSEG * j, GSEG)]],
                    obuf.at[pl.ds(GSEG * j, GSEG)],
                    sem,
                )
                for j in range(NSEGE)
            ]
            for cp in copies:
                cp.wait()

            pltpu.sync_copy(
                obuf, out_hbm.at[pl.ds(C * cstart, C * CHUNK)]
            )
            return carry

        lax.fori_loop(0, NCHUNK, chunk_body, 0)

    return k(coords_flat, image_flat)


def kernel(coordinates, image):
    coords_flat = jnp.minimum(coordinates.reshape(-1), jnp.float32(4 * W))
    out = _sc_lookup(coords_flat, image.reshape(-1))
    return (out * jnp.float32(1.0 / 255.0)).reshape(N, C)


# speedup vs baseline: 11.5397x; 11.5397x over previous
import functools

import jax
import jax.numpy as jnp
from jax import lax
from jax.experimental import pallas as pl
from jax.experimental.pallas import tpu as pltpu
from jax.experimental.pallas import tpu_sc as plsc

H = 512
W = 512
C = 3
HW = H * W
N = 2097152

LANES = 16
NC = 2
NS = 16
NW = NC * NS

CPT = N // NW
CHUNK = 2048
NCHUNK = CPT // CHUNK
GSEG = 128
NSEG = CHUNK // GSEG


def _sc_lookup(ys, xs, img_planar):
    mesh = plsc.VectorSubcoreMesh(core_axis_name="c", subcore_axis_name="s")

    plane = jax.ShapeDtypeStruct((N,), jnp.float32)

    @functools.partial(
        pl.kernel,
        mesh=mesh,
        out_type=(plane, plane, plane),
        compiler_params=pltpu.CompilerParams(use_tc_tiling_on_sc=False),
        scratch_types=[
            pltpu.VMEM((CHUNK,), jnp.float32),
            pltpu.VMEM((CHUNK,), jnp.float32),
            pltpu.VMEM((C * CHUNK,), jnp.int32),
            pltpu.VMEM((C * CHUNK,), jnp.float32),
            pltpu.SemaphoreType.DMA,
        ],
    )
    def k(ys_hbm, xs_hbm, img_hbm, out_r, out_g, out_b, ybuf, xbuf, ebuf,
          obuf, sem):
        wid = lax.axis_index("s") * NC + lax.axis_index("c")
        base = wid * CPT

        def chunk_body(g, carry):
            cstart = base + g * CHUNK
            pltpu.sync_copy(ys_hbm.at[pl.ds(cstart, CHUNK)], ybuf)
            pltpu.sync_copy(xs_hbm.at[pl.ds(cstart, CHUNK)], xbuf)

            def grp_body(i, carry):
                for kk in range(4):
                    o = LANES * (4 * i + kk)
                    y = ybuf[pl.ds(o, LANES)].astype(jnp.int32)
                    x = xbuf[pl.ds(o, LANES)].astype(jnp.int32)
                    y = jnp.minimum(jnp.maximum(y, 0), H - 1)
                    x = jnp.minimum(jnp.maximum(x, 0), W - 1)
                    p = y * W + x
                    ebuf[pl.ds(o, LANES)] = p
                    ebuf[pl.ds(CHUNK + o, LANES)] = p + HW
                    ebuf[pl.ds(2 * CHUNK + o, LANES)] = p + 2 * HW
                return carry

            lax.fori_loop(0, CHUNK // (4 * LANES), grp_body, 0)

            copies = [
                pltpu.async_copy(
                    img_hbm.at[ebuf.at[pl.ds(GSEG * j, GSEG)]],
                    obuf.at[pl.ds(GSEG * j, GSEG)],
                    sem,
                )
                for j in range(C * NSEG)
            ]
            for cp in copies:
                cp.wait()

            for c, out_c in enumerate((out_r, out_g, out_b)):
                pltpu.sync_copy(
                    obuf.at[pl.ds(c * CHUNK, CHUNK)],
                    out_c.at[pl.ds(cstart, CHUNK)],
                )
            return carry

        lax.fori_loop(0, NCHUNK, chunk_body, 0)

    return k(ys, xs, img_planar)


def kernel(coordinates, image):
    ys = coordinates[:, 0]
    xs = coordinates[:, 1]
    img_planar = image.transpose(2, 0, 1).reshape(C * HW)
    r, g, b = _sc_lookup(ys, xs, img_planar)
    return jnp.stack([r, g, b], axis=1) * jnp.float32(1.0 / 255.0)


# speedup vs baseline: 12.5489x; 1.0875x over previous
import functools

import jax
import jax.numpy as jnp
from jax import lax
from jax.experimental import pallas as pl
from jax.experimental.pallas import tpu as pltpu
from jax.experimental.pallas import tpu_sc as plsc

H = 512
W = 512
C = 3
HW = H * W
N = 2097152

LANES = 16
NC = 2
NS = 16
NW = NC * NS

CPT = N // NW
CHUNK = 4096
NCHUNK = CPT // CHUNK
GSEG = 128
NSEG = CHUNK // GSEG


def _sc_lookup(ys, xs, img_planar):
    mesh = plsc.VectorSubcoreMesh(core_axis_name="c", subcore_axis_name="s")

    plane = jax.ShapeDtypeStruct((N,), jnp.float32)

    @functools.partial(
        pl.kernel,
        mesh=mesh,
        out_type=(plane, plane, plane),
        compiler_params=pltpu.CompilerParams(use_tc_tiling_on_sc=False),
        scratch_types=[
            pltpu.VMEM((CHUNK,), jnp.float32),
            pltpu.VMEM((CHUNK,), jnp.float32),
            pltpu.VMEM((C * CHUNK,), jnp.int32),
            pltpu.VMEM((C * CHUNK,), jnp.float32),
            pltpu.SemaphoreType.DMA,
        ],
    )
    def k(ys_hbm, xs_hbm, img_hbm, out_r, out_g, out_b, ybuf, xbuf, ebuf,
          obuf, sem):
        wid = lax.axis_index("s") * NC + lax.axis_index("c")
        base = wid * CPT

        def chunk_body(g, carry):
            cstart = base + g * CHUNK
            pltpu.sync_copy(ys_hbm.at[pl.ds(cstart, CHUNK)], ybuf)
            pltpu.sync_copy(xs_hbm.at[pl.ds(cstart, CHUNK)], xbuf)

            def grp_body(i, carry):
                for kk in range(4):
                    o = LANES * (4 * i + kk)
                    y = ybuf[pl.ds(o, LANES)].astype(jnp.int32)
                    x = xbuf[pl.ds(o, LANES)].astype(jnp.int32)
                    y = jnp.minimum(jnp.maximum(y, 0), H - 1)
                    x = jnp.minimum(jnp.maximum(x, 0), W - 1)
                    p = y * W + x
                    ebuf[pl.ds(o, LANES)] = p
                    ebuf[pl.ds(CHUNK + o, LANES)] = p + HW
                    ebuf[pl.ds(2 * CHUNK + o, LANES)] = p + 2 * HW
                return carry

            lax.fori_loop(0, CHUNK // (4 * LANES), grp_body, 0)

            copies = [
                pltpu.async_copy(
                    img_hbm.at[ebuf.at[pl.ds(GSEG * j, GSEG)]],
                    obuf.at[pl.ds(GSEG * j, GSEG)],
                    sem,
                )
                for j in range(C * NSEG)
            ]
            for cp in copies:
                cp.wait()

            for c, out_c in enumerate((out_r, out_g, out_b)):
                pltpu.sync_copy(
                    obuf.at[pl.ds(c * CHUNK, CHUNK)],
                    out_c.at[pl.ds(cstart, CHUNK)],
                )
            return carry

        lax.fori_loop(0, NCHUNK, chunk_body, 0)

    return k(ys, xs, img_planar)


def kernel(coordinates, image):
    ys = coordinates[:, 0]
    xs = coordinates[:, 1]
    img_planar = image.transpose(2, 0, 1).reshape(C * HW)
    r, g, b = _sc_lookup(ys, xs, img_planar)
    return jnp.stack([r, g, b], axis=1) * jnp.float32(1.0 / 255.0)


# speedup vs baseline: 13.0018x; 1.0361x over previous
import functools

import jax
import jax.numpy as jnp
from jax import lax
from jax.experimental import pallas as pl
from jax.experimental.pallas import tpu as pltpu
from jax.experimental.pallas import tpu_sc as plsc

H = 512
W = 512
C = 3
HW = H * W
N = 2097152

LANES = 16
NC = 2
NS = 16
NW = NC * NS

CPT = N // NW
CHUNK = 8192
NCHUNK = CPT // CHUNK
GSEG = 128
NSEG = CHUNK // GSEG


def _sc_lookup(ys, xs, img_planar):
    mesh = plsc.VectorSubcoreMesh(core_axis_name="c", subcore_axis_name="s")

    plane = jax.ShapeDtypeStruct((N,), jnp.float32)

    @functools.partial(
        pl.kernel,
        mesh=mesh,
        out_type=(plane, plane, plane),
        compiler_params=pltpu.CompilerParams(use_tc_tiling_on_sc=False),
        scratch_types=[
            pltpu.VMEM((CHUNK,), jnp.float32),
            pltpu.VMEM((CHUNK,), jnp.float32),
            pltpu.VMEM((C * CHUNK,), jnp.int32),
            pltpu.VMEM((C * CHUNK,), jnp.float32),
            pltpu.SemaphoreType.DMA,
        ],
    )
    def k(ys_hbm, xs_hbm, img_hbm, out_r, out_g, out_b, ybuf, xbuf, ebuf,
          obuf, sem):
        wid = lax.axis_index("s") * NC + lax.axis_index("c")
        base = wid * CPT

        def chunk_body(g, carry):
            cstart = base + g * CHUNK
            pltpu.sync_copy(ys_hbm.at[pl.ds(cstart, CHUNK)], ybuf)
            pltpu.sync_copy(xs_hbm.at[pl.ds(cstart, CHUNK)], xbuf)

            def grp_body(i, carry):
                for kk in range(4):
                    o = LANES * (4 * i + kk)
                    y = ybuf[pl.ds(o, LANES)].astype(jnp.int32)
                    x = xbuf[pl.ds(o, LANES)].astype(jnp.int32)
                    y = jnp.minimum(jnp.maximum(y, 0), H - 1)
                    x = jnp.minimum(jnp.maximum(x, 0), W - 1)
                    p = y * W + x
                    ebuf[pl.ds(o, LANES)] = p
                    ebuf[pl.ds(CHUNK + o, LANES)] = p + HW
                    ebuf[pl.ds(2 * CHUNK + o, LANES)] = p + 2 * HW
                return carry

            lax.fori_loop(0, CHUNK // (4 * LANES), grp_body, 0)

            copies = [
                pltpu.async_copy(
                    img_hbm.at[ebuf.at[pl.ds(GSEG * j, GSEG)]],
                    obuf.at[pl.ds(GSEG * j, GSEG)],
                    sem,
                )
                for j in range(C * NSEG)
            ]
            for cp in copies:
                cp.wait()

            for c, out_c in enumerate((out_r, out_g, out_b)):
                pltpu.sync_copy(
                    obuf.at[pl.ds(c * CHUNK, CHUNK)],
                    out_c.at[pl.ds(cstart, CHUNK)],
                )
            return carry

        lax.fori_loop(0, NCHUNK, chunk_body, 0)

    return k(ys, xs, img_planar)


def kernel(coordinates, image):
    ys = coordinates[:, 0]
    xs = coordinates[:, 1]
    img_planar = image.transpose(2, 0, 1).reshape(C * HW)
    r, g, b = _sc_lookup(ys, xs, img_planar)
    return jnp.stack([r, g, b], axis=1) * jnp.float32(1.0 / 255.0)


# speedup vs baseline: 13.9010x; 1.0692x over previous
import functools

import jax
import jax.numpy as jnp
from jax import lax
from jax.experimental import pallas as pl
from jax.experimental.pallas import tpu as pltpu
from jax.experimental.pallas import tpu_sc as plsc

H = 512
W = 512
C = 3
HW = H * W
N = 2097152

LANES = 16
NC = 2
NS = 16
NW = NC * NS

CPT = N // NW
CHUNK = 4096
NCHUNK = CPT // CHUNK
GSEG = 128
NSEG = CHUNK // GSEG
GPS = GSEG // LANES

YX_BYTES = 2 * CHUNK * 4
GATHER_BYTES = C * CHUNK * 4
OUT_BYTES = C * CHUNK * 4


def _sc_lookup(ys, xs, img_planar):
    mesh = plsc.VectorSubcoreMesh(core_axis_name="c", subcore_axis_name="s")

    plane = jax.ShapeDtypeStruct((N,), jnp.float32)

    @functools.partial(
        pl.kernel,
        mesh=mesh,
        out_type=(plane, plane, plane),
        compiler_params=pltpu.CompilerParams(use_tc_tiling_on_sc=False),
        scratch_types=[
            pltpu.VMEM((2, CHUNK), jnp.float32),
            pltpu.VMEM((2, CHUNK), jnp.float32),
            pltpu.VMEM((2, C * CHUNK), jnp.int32),
            pltpu.VMEM((2, C * CHUNK), jnp.float32),
            pltpu.SemaphoreType.DMA,
            pltpu.SemaphoreType.DMA,
            pltpu.SemaphoreType.DMA,
        ],
    )
    def k(ys_hbm, xs_hbm, img_hbm, out_r, out_g, out_b, ybuf, xbuf, ebuf,
          obuf, ysem, gsem, osem):
        wid = lax.axis_index("s") * NC + lax.axis_index("c")
        base = wid * CPT

        def fetch_yx(g, b):
            cstart = base + g * CHUNK
            pltpu.async_copy(ys_hbm.at[pl.ds(cstart, CHUNK)], ybuf.at[b],
                             ysem)
            pltpu.async_copy(xs_hbm.at[pl.ds(cstart, CHUNK)], xbuf.at[b],
                             ysem)

        fetch_yx(0, 0)

        def half(g2, g, b):
            cstart = base + g * CHUNK
            eb = ebuf.at[b]
            ob = obuf.at[b]

            @pl.when(g2 >= 1)
            def _():
                for c in range(C):
                    pltpu.make_async_copy(
                        ob.at[pl.ds(c * CHUNK, CHUNK)],
                        out_r.at[pl.ds(base, CHUNK)],
                        osem,
                    ).wait()

            pltpu.make_async_copy(
                ys_hbm.at[pl.ds(base, CHUNK)], ybuf.at[b], ysem
            ).wait()
            pltpu.make_async_copy(
                xs_hbm.at[pl.ds(base, CHUNK)], xbuf.at[b], ysem
            ).wait()

            def seg_body(j, carry):
                for kk in range(GPS):
                    o = GSEG * j + LANES * kk
                    y = ybuf[b, pl.ds(o, LANES)].astype(jnp.int32)
                    x = xbuf[b, pl.ds(o, LANES)].astype(jnp.int32)
                    y = jnp.minimum(jnp.maximum(y, 0), H - 1)
                    x = jnp.minimum(jnp.maximum(x, 0), W - 1)
                    p = y * W + x
                    eb[pl.ds(o, LANES)] = p
                    eb[pl.ds(CHUNK + o, LANES)] = p + HW
                    eb[pl.ds(2 * CHUNK + o, LANES)] = p + 2 * HW
                for c in range(C):
                    off = c * CHUNK + GSEG * j
                    pltpu.async_copy(
                        img_hbm.at[eb.at[pl.ds(off, GSEG)]],
                        ob.at[pl.ds(off, GSEG)],
                        gsem,
                    )
                return carry

            lax.fori_loop(0, NSEG, seg_body, 0)

            return cstart, ob

        def pair_body(g2, carry):
            cstart, ob = half(g2, 2 * g2, 0)
            fetch_yx(2 * g2 + 1, 1)
            pltpu.make_async_copy(
                img_hbm.at[pl.ds(0, C * CHUNK)], ob, gsem
            ).wait()
            for c, out_c in enumerate((out_r, out_g, out_b)):
                pltpu.async_copy(
                    ob.at[pl.ds(c * CHUNK, CHUNK)],
                    out_c.at[pl.ds(cstart, CHUNK)],
                    osem,
                )

            cstart, ob = half(g2, 2 * g2 + 1, 1)

            @pl.when(g2 < NCHUNK // 2 - 1)
            def _():
                fetch_yx(2 * g2 + 2, 0)

            pltpu.make_async_copy(
                img_hbm.at[pl.ds(0, C * CHUNK)], ob, gsem
            ).wait()
            for c, out_c in enumerate((out_r, out_g, out_b)):
                pltpu.async_copy(
                    ob.at[pl.ds(c * CHUNK, CHUNK)],
                    out_c.at[pl.ds(cstart, CHUNK)],
                    osem,
                )
            return carry

        lax.fori_loop(0, NCHUNK // 2, pair_body, 0)

        for b in range(2):
            for c in range(C):
                pltpu.make_async_copy(
                    obuf.at[b, pl.ds(c * CHUNK, CHUNK)],
                    out_r.at[pl.ds(base, CHUNK)],
                    osem,
                ).wait()

    return k(ys, xs, img_planar)


def kernel(coordinates, image):
    ys = coordinates[:, 0]
    xs = coordinates[:, 1]
    img_planar = image.transpose(2, 0, 1).reshape(C * HW)
    r, g, b = _sc_lookup(ys, xs, img_planar)
    return jnp.stack([r, g, b], axis=1) * jnp.float32(1.0 / 255.0)


# speedup vs baseline: 31.7213x; 2.2819x over previous
import functools

import jax
import jax.numpy as jnp
from jax import lax
from jax.experimental import pallas as pl
from jax.experimental.pallas import tpu as pltpu
from jax.experimental.pallas import tpu_sc as plsc

H = 512
W = 512
C = 3
HW = H * W
N = 2097152

LANES = 16
NC = 2
NS = 16
NW = NC * NS

CPT = N // NW
CHUNK = 4096
NCHUNK = CPT // CHUNK
GSEG = 128
NSEG = CHUNK // GSEG
GPS = GSEG // LANES

YX_BYTES = 2 * CHUNK * 4
GATHER_BYTES = C * CHUNK * 4
OUT_BYTES = C * CHUNK * 4


def _sc_lookup(ys, xs, img_planar):
    mesh = plsc.VectorSubcoreMesh(core_axis_name="c", subcore_axis_name="s")

    plane = jax.ShapeDtypeStruct((N,), jnp.float32)

    @functools.partial(
        pl.kernel,
        mesh=mesh,
        out_type=(plane, plane, plane),
        compiler_params=pltpu.CompilerParams(use_tc_tiling_on_sc=False),
        scratch_types=[
            pltpu.VMEM((2, CHUNK), jnp.float32),
            pltpu.VMEM((2, CHUNK), jnp.float32),
            pltpu.VMEM((2, C * CHUNK), jnp.int32),
            pltpu.VMEM((2, C * CHUNK), jnp.float32),
            pltpu.VMEM_SHARED((C * HW,), jnp.float32),
            pltpu.SemaphoreType.DMA,
            pltpu.SemaphoreType.DMA,
            pltpu.SemaphoreType.DMA,
        ],
    )
    def k(ys_hbm, xs_hbm, img_hbm, out_r, out_g, out_b, ybuf, xbuf, ebuf,
          obuf, simg, ysem, gsem, osem):
        wid = lax.axis_index("s") * NC + lax.axis_index("c")
        base = wid * CPT

        sid = lax.axis_index("s")
        ISL = C * HW // NS
        pltpu.sync_copy(
            img_hbm.at[pl.ds(sid * ISL, ISL)], simg.at[pl.ds(sid * ISL, ISL)]
        )
        plsc.subcore_barrier()

        def fetch_yx(g, b):
            cstart = base + g * CHUNK
            pltpu.async_copy(ys_hbm.at[pl.ds(cstart, CHUNK)], ybuf.at[b],
                             ysem)
            pltpu.async_copy(xs_hbm.at[pl.ds(cstart, CHUNK)], xbuf.at[b],
                             ysem)

        fetch_yx(0, 0)

        def half(g2, g, b):
            cstart = base + g * CHUNK
            eb = ebuf.at[b]
            ob = obuf.at[b]

            @pl.when(g2 >= 1)
            def _():
                for c in range(C):
                    pltpu.make_async_copy(
                        ob.at[pl.ds(c * CHUNK, CHUNK)],
                        out_r.at[pl.ds(base, CHUNK)],
                        osem,
                    ).wait()

            pltpu.make_async_copy(
                ys_hbm.at[pl.ds(base, CHUNK)], ybuf.at[b], ysem
            ).wait()
            pltpu.make_async_copy(
                xs_hbm.at[pl.ds(base, CHUNK)], xbuf.at[b], ysem
            ).wait()

            def seg_body(j, carry):
                for kk in range(GPS):
                    o = GSEG * j + LANES * kk
                    y = ybuf[b, pl.ds(o, LANES)].astype(jnp.int32)
                    x = xbuf[b, pl.ds(o, LANES)].astype(jnp.int32)
                    y = jnp.minimum(jnp.maximum(y, 0), H - 1)
                    x = jnp.minimum(jnp.maximum(x, 0), W - 1)
                    p = y * W + x
                    eb[pl.ds(o, LANES)] = p
                    eb[pl.ds(CHUNK + o, LANES)] = p + HW
                    eb[pl.ds(2 * CHUNK + o, LANES)] = p + 2 * HW
                for c in range(C):
                    off = c * CHUNK + GSEG * j
                    pltpu.async_copy(
                        simg.at[eb.at[pl.ds(off, GSEG)]],
                        ob.at[pl.ds(off, GSEG)],
                        gsem,
                    )
                return carry

            lax.fori_loop(0, NSEG, seg_body, 0)

            return cstart, ob

        def pair_body(g2, carry):
            cstart, ob = half(g2, 2 * g2, 0)
            fetch_yx(2 * g2 + 1, 1)
            pltpu.make_async_copy(
                img_hbm.at[pl.ds(0, C * CHUNK)], ob, gsem
            ).wait()
            for c, out_c in enumerate((out_r, out_g, out_b)):
                pltpu.async_copy(
                    ob.at[pl.ds(c * CHUNK, CHUNK)],
                    out_c.at[pl.ds(cstart, CHUNK)],
                    osem,
                )

            cstart, ob = half(g2, 2 * g2 + 1, 1)

            @pl.when(g2 < NCHUNK // 2 - 1)
            def _():
                fetch_yx(2 * g2 + 2, 0)

            pltpu.make_async_copy(
                img_hbm.at[pl.ds(0, C * CHUNK)], ob, gsem
            ).wait()
            for c, out_c in enumerate((out_r, out_g, out_b)):
                pltpu.async_copy(
                    ob.at[pl.ds(c * CHUNK, CHUNK)],
                    out_c.at[pl.ds(cstart, CHUNK)],
                    osem,
                )
            return carry

        lax.fori_loop(0, NCHUNK // 2, pair_body, 0)

        for b in range(2):
            for c in range(C):
                pltpu.make_async_copy(
                    obuf.at[b, pl.ds(c * CHUNK, CHUNK)],
                    out_r.at[pl.ds(base, CHUNK)],
                    osem,
                ).wait()

    return k(ys, xs, img_planar)


def kernel(coordinates, image):
    ys = coordinates[:, 0]
    xs = coordinates[:, 1]
    img_planar = image.transpose(2, 0, 1).reshape(C * HW)
    r, g, b = _sc_lookup(ys, xs, img_planar)
    return jnp.stack([r, g, b], axis=1) * jnp.float32(1.0 / 255.0)
